# baseline (speedup 1.0000x reference)
import functools

import jax
import jax.numpy as jnp
from jax import lax
from jax.experimental import pallas as pl
from jax.experimental.pallas import tpu as pltpu


def _fea_body(x1_ref, x2_ref, w0_ref, w1_ref, b_ref, o_ref):
    acc = jnp.dot(x1_ref[...], w0_ref[...], preferred_element_type=jnp.float32)
    acc += jnp.dot(x2_ref[...], w1_ref[...], preferred_element_type=jnp.float32)
    o_ref[...] = acc + b_ref[...][None, :]


def _fea_matmul(x1, x2, W, b, bm):
    n, f = x1.shape
    out_f = W.shape[0]
    w0t = W[:, 0, :].T
    w1t = W[:, 1, :].T
    grid = (n // bm,)
    return pl.pallas_call(
        _fea_body,
        grid=grid,
        in_specs=[
            pl.BlockSpec((bm, f), lambda i: (i, 0)),
            pl.BlockSpec((bm, f), lambda i: (i, 0)),
            pl.BlockSpec((f, out_f), lambda i: (0, 0)),
            pl.BlockSpec((f, out_f), lambda i: (0, 0)),
            pl.BlockSpec((out_f,), lambda i: (0,)),
        ],
        out_specs=pl.BlockSpec((bm, out_f), lambda i: (i, 0)),
        out_shape=jax.ShapeDtypeStruct((n, out_f), jnp.float32),
    )(x1, x2, w0t, w1t, b)


def _score_body(c_ref, d_ref, o_ref):
    o_ref[...] = lax.dot_general(
        c_ref[...], d_ref[...], (((1,), (1,)), ((), ())),
        preferred_element_type=jnp.float32)


def _score_matmul(cf, df, bm, bn):
    nc, k = cf.shape
    nd = df.shape[0]
    grid = (nc // bm, nd // bn)
    return pl.pallas_call(
        _score_body,
        grid=grid,
        in_specs=[
            pl.BlockSpec((bm, k), lambda i, j: (i, 0)),
            pl.BlockSpec((bn, k), lambda i, j: (j, 0)),
        ],
        out_specs=pl.BlockSpec((bm, bn), lambda i, j: (i, j)),
        out_shape=jax.ShapeDtypeStruct((nc, nd), jnp.float32),
    )(cf, df)


def _gcn(x, ei, ew, p):
    n = x.shape[0]
    sl = jnp.arange(n)
    row = jnp.concatenate([ei[0], sl])
    col = jnp.concatenate([ei[1], sl])
    w = jnp.concatenate([ew, jnp.ones((n,), x.dtype)])
    deg = jax.ops.segment_sum(w, col, num_segments=n)
    dinv = lax.rsqrt(jnp.maximum(deg, 1e-12))
    norm = dinv[row] * w * dinv[col]
    xw = x @ p["W"]
    out = jax.ops.segment_sum(norm[:, None] * xw[row], col, num_segments=n)
    return out + p["b"]


def _gat(x, ei, ea, p, h):
    n, f = x.shape
    sl = jnp.arange(n)
    row = jnp.concatenate([ei[0], sl])
    col = jnp.concatenate([ei[1], sl])
    eat = jnp.concatenate([ea, jnp.full((n,), jnp.mean(ea), x.dtype)])
    xh = (x @ p["W"]).reshape(n, h, f)
    eh = (eat[:, None] * p["lin_edge"]).reshape(-1, h, f)
    a = ((xh * p["att_src"][None]).sum(-1)[row]
         + (xh * p["att_dst"][None]).sum(-1)[col]
         + (eh * p["att_edge"][None]).sum(-1))
    a = jax.nn.leaky_relu(a, 0.2)
    amax = jax.ops.segment_max(a, col, num_segments=n)
    amax = jnp.where(jnp.isfinite(amax), amax, 0.0)
    ex = jnp.exp(a - amax[col])
    den = jax.ops.segment_sum(ex, col, num_segments=n)
    att = ex / (den[col] + 1e-16)
    out = jax.ops.segment_sum(att[:, :, None] * xh[row], col, num_segments=n)
    return out.mean(axis=1) + p["b"]


def kernel(circ_edge_index, circ_adj, dis_edge_index, dis_adj, x_cir, x_dis, params):
    fcir = x_cir.shape[1]
    fdis = x_dis.shape[1]

    ew_c = circ_adj[circ_edge_index[0], circ_edge_index[1]]
    ew_d = dis_adj[dis_edge_index[0], dis_edge_index[1]]

    gat1 = _gat(x_cir, circ_edge_index, ew_c, params["gat_cir1"], 2)
    gcn1 = _gcn(x_cir, circ_edge_index, ew_c, params["gcn_cir1"])
    x_cir_f1 = jax.nn.relu((gcn1 + gat1) / 2)
    l2 = (_gcn(x_cir_f1, circ_edge_index, ew_c, params["gcn_cir2"])
          + _gat(x_cir_f1, circ_edge_index, ew_c, params["gat_cir2"], 1))
    x_cir_f2 = jax.nn.relu(l2 / 2)

    d1 = (_gcn(x_dis, dis_edge_index, ew_d, params["gcn_dis1"])
          + _gat(x_dis, dis_edge_index, ew_d, params["gat_dis1"], 4))
    x_dis_f1 = jax.nn.relu(d1 / 2)
    d2 = (_gcn(x_dis_f1, dis_edge_index, ew_d, params["gcn_dis2"])
          + _gat(x_dis_f1, dis_edge_index, ew_d, params["gat_dis1"], 4))
    x_dis_f2 = jax.nn.relu(d2 / 2)

    dis_fea = _fea_matmul(x_dis_f1, x_dis_f2, params["cnn_dis"]["W"],
                          params["cnn_dis"]["b"], bm=1024)
    cir_fea = _fea_matmul(x_cir_f1, x_cir_f2, params["cnn_cir"]["W"],
                          params["cnn_cir"]["b"], bm=1000)
    scores = _score_matmul(cir_fea, dis_fea, bm=1000, bn=1024)
    return (scores, cir_fea, dis_fea)


# speedup vs baseline: 2.5437x; 2.5437x over previous
import functools

import jax
import jax.numpy as jnp
from jax import lax
from jax.experimental import pallas as pl
from jax.experimental.pallas import tpu as pltpu
from jax.experimental.pallas import tpu_sc as plsc

_NC = 2
_NS = 16
_NW = _NC * _NS


def _wid():
    return lax.axis_index("s") * _NC + lax.axis_index("c")


def _range_split(total, parts, wid):
    q, r = total // parts, total % parts
    cnt = q + jnp.where(wid < r, 1, 0)
    base = wid * q + jnp.minimum(wid, r)
    return base, cnt


def _sc_edge_gather(adj, row, col):
    n = adj.shape[0]
    e = row.shape[0]
    nb = e // 128
    mesh = plsc.VectorSubcoreMesh(core_axis_name="c", subcore_axis_name="s")

    @functools.partial(
        pl.kernel,
        out_type=jax.ShapeDtypeStruct((e,), jnp.float32),
        mesh=mesh,
        scratch_types=[
            pltpu.VMEM((128,), jnp.int32),
            pltpu.VMEM((128,), jnp.int32),
            pltpu.VMEM((128,), jnp.int32),
            pltpu.VMEM((128,), jnp.float32),
            pltpu.SemaphoreType.DMA,
        ],
    )
    def k(adj_ref, row_ref, col_ref, out_ref, rowb, colb, idxb, ewb, sem):
        base_b, cnt = _range_split(nb, _NW, _wid())

        def body(i, carry):
            base = (base_b + i) * 128
            pltpu.sync_copy(row_ref.at[pl.ds(base, 128)], rowb)
            pltpu.sync_copy(col_ref.at[pl.ds(base, 128)], colb)
            for g in range(8):
                sl = pl.ds(g * 16, 16)
                idxb[sl] = rowb[sl] * n + colb[sl]
            pltpu.async_copy(adj_ref.at[idxb], ewb, sem).wait()
            pltpu.sync_copy(ewb, out_ref.at[pl.ds(base, 128)])
            return carry

        lax.fori_loop(0, cnt, body, 0)

    return k(adj.reshape(n * n), row, col)


def _iota16():
    return lax.iota(jnp.int32, 16)


def _sc_edge_pass(row, col, ew, ssrc_c, sdst_c, consts, heads, n):
    e = row.shape[0]
    nb = e // 128
    gat = heads > 0
    h_eff = heads if gat else 1
    hn = h_eff * n
    mesh = plsc.VectorSubcoreMesh(core_axis_name="c", subcore_axis_name="s")

    out_type = [jax.ShapeDtypeStruct((2 * hn,), jnp.float32)]
    if gat:
        out_type.append(jax.ShapeDtypeStruct((heads * e,), jnp.float32))

    scratch = [
        pltpu.VMEM((128,), jnp.int32),
        pltpu.VMEM((128,), jnp.int32),
        pltpu.VMEM((128,), jnp.float32),
        pltpu.VMEM((128,), jnp.int32),
        pltpu.VMEM((128,), jnp.float32),
        pltpu.VMEM_SHARED((hn,), jnp.float32),
        pltpu.SemaphoreType.DMA,
    ]
    if gat:
        scratch += [
            pltpu.VMEM((n * heads,), jnp.float32),
            pltpu.VMEM((n * heads,), jnp.float32),
            pltpu.VMEM((heads * 32,), jnp.float32),
        ]

    @functools.partial(pl.kernel, out_type=tuple(out_type), mesh=mesh,
                       scratch_types=scratch,
                       compiler_params=pltpu.CompilerParams(
                           needs_layout_passes=False))
    def k(*refs):
        if gat:
            (row_ref, col_ref, ew_ref, ssrc_ref, sdst_ref, cst_ref,
             den_out, ex_out,
             rowb, colb, ewb, idxb, exvb, den, sem,
             ssrct, sdstt, cb) = refs
        else:
            (row_ref, col_ref, ew_ref, den_out,
             rowb, colb, ewb, idxb, exvb, den, sem) = refs

        core = lax.axis_index("c")
        sid = lax.axis_index("s")
        wid = sid * _NC + core

        if gat:
            pltpu.sync_copy(ssrc_ref, ssrct)
            pltpu.sync_copy(sdst_ref, sdstt)
            pltpu.sync_copy(cst_ref, cb)

        z16 = jnp.zeros((16,), jnp.float32)
        for g in range(8):
            exvb[pl.ds(g * 16, 16)] = z16
        zbase, zcnt = _range_split(hn // 16, _NS, sid)

        def zstripe(i, c):
            pltpu.sync_copy(exvb.at[pl.ds(0, 16)],
                            den.at[pl.ds((zbase + i) * 16, 16)])
            return c

        lax.fori_loop(0, zcnt, zstripe, 0)
        plsc.subcore_barrier()

        base_b, cnt = _range_split(nb, _NW, wid)

        def body(i, carry):
            base = (base_b + i) * 128
            pltpu.sync_copy(row_ref.at[pl.ds(base, 128)], rowb)
            pltpu.sync_copy(col_ref.at[pl.ds(base, 128)], colb)
            pltpu.sync_copy(ew_ref.at[pl.ds(base, 128)], ewb)
            if gat:
                for h in range(heads):
                    for g in range(8):
                        sl = pl.ds(g * 16, 16)
                        rv = rowb[sl]
                        cv = colb[sl]
                        wv = ewb[sl]
                        ssv = plsc.load_gather(ssrct, [rv * heads + h])
                        sdv = plsc.load_gather(sdstt, [cv * heads + h])
                        cev = cb[pl.ds(h * 32, 16)]
                        mv = cb[pl.ds(h * 32 + 16, 16)]
                        av = ssv + sdv + wv * cev
                        av = jnp.where(av >= 0, av, av * 0.2)
                        exv = jnp.exp(av - mv)
                        exvb[sl] = exv
                        idxb[sl] = cv * heads + h
                    pltpu.sync_copy(exvb, den.at[idxb], add=True)
                    pltpu.sync_copy(exvb, ex_out.at[pl.ds(h * e + base, 128)])
            else:
                for g in range(8):
                    sl = pl.ds(g * 16, 16)
                    idxb[sl] = colb[sl]
                pltpu.sync_copy(ewb, den.at[idxb], add=True)
            return carry

        lax.fori_loop(0, cnt, body, 0)
        plsc.subcore_barrier()

        def dump(i, c):
            pltpu.sync_copy(den.at[pl.ds((zbase + i) * 16, 16)],
                            exvb.at[pl.ds(0, 16)])
            pltpu.sync_copy(exvb.at[pl.ds(0, 16)],
                            den_out.at[pl.ds(core * hn + (zbase + i) * 16, 16)])
            return c

        lax.fori_loop(0, zcnt, dump, 0)

    if gat:
        return k(row, col, ew, ssrc_c, sdst_c, consts)
    return k(row, col, ew)[0]


def _row_spec(bm, width):
    return pl.BlockSpec((bm, width), lambda i: (i, 0))


def _full_spec(shape):
    return pl.BlockSpec(shape, lambda i: tuple(0 for _ in shape))


def _mm(x, w, bm):
    n, kdim = x.shape
    m = w.shape[1]

    def body(x_ref, w_ref, o_ref):
        o_ref[...] = jnp.dot(x_ref[...], w_ref[...],
                             preferred_element_type=jnp.float32)

    return pl.pallas_call(
        body, grid=(n // bm,),
        in_specs=[_row_spec(bm, kdim), _full_spec((kdim, m))],
        out_specs=_row_spec(bm, m),
        out_shape=jax.ShapeDtypeStruct((n, m), jnp.float32))(x, w)


def _tc_gat_proj(x, w, asrcc, adstc, bm):
    n, f = x.shape
    hf = w.shape[1]

    def body(x_ref, w_ref, a_ref, d_ref, xh_ref, ss_ref, sd_ref, ms_ref, md_ref):
        i = pl.program_id(0)
        xh = jnp.dot(x_ref[...], w_ref[...], preferred_element_type=jnp.float32)
        xh_ref[...] = xh
        ss = jnp.dot(xh, a_ref[...], preferred_element_type=jnp.float32)
        sd = jnp.dot(xh, d_ref[...], preferred_element_type=jnp.float32)
        ss_ref[...] = ss
        sd_ref[...] = sd
        msb = jnp.broadcast_to(jnp.max(ss, axis=0, keepdims=True), (8, 128))
        mdb = jnp.broadcast_to(jnp.max(sd, axis=0, keepdims=True), (8, 128))

        @pl.when(i == 0)
        def _():
            ms_ref[...] = msb
            md_ref[...] = mdb

        @pl.when(i > 0)
        def _():
            ms_ref[...] = jnp.maximum(ms_ref[...], msb)
            md_ref[...] = jnp.maximum(md_ref[...], mdb)

    return pl.pallas_call(
        body, grid=(n // bm,),
        in_specs=[_row_spec(bm, f), _full_spec((f, hf)),
                  _full_spec((hf, 128)), _full_spec((hf, 128))],
        out_specs=[_row_spec(bm, hf), _row_spec(bm, 128), _row_spec(bm, 128),
                   _full_spec((8, 128)), _full_spec((8, 128))],
        out_shape=[jax.ShapeDtypeStruct((n, hf), jnp.float32),
                   jax.ShapeDtypeStruct((n, 128), jnp.float32),
                   jax.ShapeDtypeStruct((n, 128), jnp.float32),
                   jax.ShapeDtypeStruct((8, 128), jnp.float32),
                   jax.ShapeDtypeStruct((8, 128), jnp.float32)])(
        x, w, asrcc, adstc)


def _tc_deg_fin(d0, d1, bm):
    n = d0.shape[0]

    def body(a_ref, b_ref, di_ref, sm_ref):
        i = pl.program_id(0)
        deg = a_ref[...] + b_ref[...] + 1.0
        di_ref[...] = lax.rsqrt(jnp.maximum(deg, 1e-12))
        s = jnp.broadcast_to(jnp.sum(deg[:, 0:1]), (8, 128))

        @pl.when(i == 0)
        def _():
            sm_ref[...] = s

        @pl.when(i > 0)
        def _():
            sm_ref[...] = sm_ref[...] + s

    return pl.pallas_call(
        body, grid=(n // bm,),
        in_specs=[_row_spec(bm, 128), _row_spec(bm, 128)],
        out_specs=[_row_spec(bm, 128), _full_spec((8, 128))],
        out_shape=[jax.ShapeDtypeStruct((n, 128), jnp.float32),
                   jax.ShapeDtypeStruct((8, 128), jnp.float32)])(d0, d1)


def _tc_gat_fin(ssrc, sdst, d0, d1, cst, heads, bm):
    n = ssrc.shape[0]

    def body(ss_ref, sd_ref, a_ref, b_ref, c_ref, rd_ref, sc_ref):
        cem = c_ref[0:1, :]
        mrow = c_ref[1:2, :]
        aself = ss_ref[...] + sd_ref[...] + cem
        aself = jnp.where(aself >= 0, aself, aself * 0.2)
        exself = jnp.exp(aself - mrow)
        dent = a_ref[...] + b_ref[...] + exself
        rden = (1.0 / (dent + 1e-16)) * (1.0 / heads)
        rd_ref[...] = rden
        sc_ref[...] = exself * rden

    return pl.pallas_call(
        body, grid=(n // bm,),
        in_specs=[_row_spec(bm, 128), _row_spec(bm, 128),
                  _row_spec(bm, 128), _row_spec(bm, 128),
                  _full_spec((8, 128))],
        out_specs=[_row_spec(bm, 128), _row_spec(bm, 128)],
        out_shape=[jax.ShapeDtypeStruct((n, 128), jnp.float32),
                   jax.ShapeDtypeStruct((n, 128), jnp.float32)])(
        ssrc, sdst, d0, d1, cst)


def _tc_scale(dinv, xw, bm):
    n, f = xw.shape

    def body(di_ref, xw_ref, o_ref):
        o_ref[...] = di_ref[:, 0:1] * xw_ref[...]

    return pl.pallas_call(
        body, grid=(n // bm,),
        in_specs=[_row_spec(bm, 128), _row_spec(bm, f)],
        out_specs=_row_spec(bm, f),
        out_shape=jax.ShapeDtypeStruct((n, f), jnp.float32))(dinv, xw)


def _tc_epilogue(dinv, gscat, xw, bg, ascat, xh, selfc, ba, heads, bm):
    n, f = xw.shape

    def body(di_ref, gs_ref, xw_ref, bg_ref, as_ref, xh_ref, sc_ref, ba_ref,
             o_ref):
        d = di_ref[:, 0:1]
        gcn = d * gs_ref[...] + (d * d) * xw_ref[...] + bg_ref[...][None, :]
        sself = sc_ref[:, 0:1] * xh_ref[:, 0:f]
        for h in range(1, heads):
            sself += sc_ref[:, h:h + 1] * xh_ref[:, h * f:(h + 1) * f]
        gat = as_ref[...] + sself + ba_ref[...][None, :]
        o_ref[...] = jnp.maximum((gcn + gat) * 0.5, 0.0)

    return pl.pallas_call(
        body, grid=(n // bm,),
        in_specs=[_row_spec(bm, 128), _row_spec(bm, f), _row_spec(bm, f),
                  _full_spec((f,)), _row_spec(bm, f),
                  _row_spec(bm, heads * f), _row_spec(bm, 128),
                  _full_spec((f,))],
        out_specs=_row_spec(bm, f),
        out_shape=jax.ShapeDtypeStruct((n, f), jnp.float32))(
        dinv, gscat, xw, bg, ascat, xh, selfc, ba)


def _fea_body(x1_ref, x2_ref, w0_ref, w1_ref, b_ref, o_ref):
    acc = jnp.dot(x1_ref[...], w0_ref[...], preferred_element_type=jnp.float32)
    acc += jnp.dot(x2_ref[...], w1_ref[...], preferred_element_type=jnp.float32)
    o_ref[...] = acc + b_ref[...][None, :]


def _fea_matmul(x1, x2, W, b, bm):
    n, f = x1.shape
    out_f = W.shape[0]
    w0t = W[:, 0, :].T
    w1t = W[:, 1, :].T
    grid = (n // bm,)
    return pl.pallas_call(
        _fea_body,
        grid=grid,
        in_specs=[
            pl.BlockSpec((bm, f), lambda i: (i, 0)),
            pl.BlockSpec((bm, f), lambda i: (i, 0)),
            pl.BlockSpec((f, out_f), lambda i: (0, 0)),
            pl.BlockSpec((f, out_f), lambda i: (0, 0)),
            pl.BlockSpec((out_f,), lambda i: (0,)),
        ],
        out_specs=pl.BlockSpec((bm, out_f), lambda i: (i, 0)),
        out_shape=jax.ShapeDtypeStruct((n, out_f), jnp.float32),
    )(x1, x2, w0t, w1t, b)


def _score_body(c_ref, d_ref, o_ref):
    o_ref[...] = lax.dot_general(
        c_ref[...], d_ref[...], (((1,), (1,)), ((), ())),
        preferred_element_type=jnp.float32)


def _score_matmul(cf, df, bm, bn):
    nc, k = cf.shape
    nd = df.shape[0]
    grid = (nc // bm, nd // bn)
    return pl.pallas_call(
        _score_body,
        grid=grid,
        in_specs=[
            pl.BlockSpec((bm, k), lambda i, j: (i, 0)),
            pl.BlockSpec((bn, k), lambda i, j: (j, 0)),
        ],
        out_specs=pl.BlockSpec((bm, bn), lambda i, j: (i, j)),
        out_shape=jax.ShapeDtypeStruct((nc, nd), jnp.float32),
    )(cf, df)


def _selectors(att_src, att_dst, heads, f):
    eye = jnp.eye(128, dtype=jnp.float32)[:heads]
    asrcc = (att_src[:, :, None] * eye[:, None, :]).reshape(heads * f, 128)
    adstc = (att_dst[:, :, None] * eye[:, None, :]).reshape(heads * f, 128)
    return asrcc, adstc


def _layer(x, row, col, ew, dinv, mean_ea, p_gcn, p_gat, heads, bm):
    n, f = x.shape
    e = row.shape[0]

    xw = _mm(x, p_gcn["W"], bm)
    asrcc, adstc = _selectors(p_gat["att_src"], p_gat["att_dst"], heads, f)
    xh, ssrc, sdst, ms, md = _tc_gat_proj(x, p_gat["W"], asrcc, adstc, bm)

    ce = (p_gat["lin_edge"].reshape(heads, f) * p_gat["att_edge"]).sum(-1)
    ce_pad = jnp.zeros((128,), jnp.float32).at[:heads].set(ce)
    m128 = jnp.maximum(ms[0] + md[0] + jnp.maximum(ce_pad, 0.0), 0.0)
    consts = (jnp.stack([ce, m128[:heads]], axis=1)[:, :, None]
              * jnp.ones((1, 1, 16), jnp.float32)).reshape(heads * 32)

    ssrc_c = ssrc[:, :heads].reshape(n * heads)
    sdst_c = sdst[:, :heads].reshape(n * heads)
    denp, exflat = _sc_edge_pass(row, col, ew, ssrc_c, sdst_c, consts,
                                 heads, n)
    den2 = denp.reshape(2, n, heads)
    zpad = jnp.zeros((n, 128), jnp.float32)
    d0p = zpad.at[:, :heads].set(den2[0])
    d1p = zpad.at[:, :heads].set(den2[1])

    cst8 = jnp.zeros((8, 128), jnp.float32)
    cst8 = cst8.at[0].set(ce_pad * mean_ea).at[1].set(m128)
    rden, selfc = _tc_gat_fin(ssrc, sdst, d0p, d1p, cst8, heads, bm)
    rden_c = rden[:, :heads].reshape(n * heads)

    y = _tc_scale(dinv, xw, bm)
    gfull = jax.ops.segment_sum(ew[:, None] * y[row], col, num_segments=n)
    ex2 = exflat.reshape(heads, e)
    rdf = rden[:, :heads]
    afull = 0.0
    for h in range(heads):
        ca = ex2[h] * rdf[col, h]
        afull += jax.ops.segment_sum(
            ca[:, None] * xh[:, h * f:(h + 1) * f][row], col, num_segments=n)

    return _tc_epilogue(dinv, gfull, xw, p_gcn["b"],
                        afull, xh, selfc, p_gat["b"], heads, bm)


def kernel(circ_edge_index, circ_adj, dis_edge_index, dis_adj, x_cir, x_dis, params):
    nc, nd = x_cir.shape[0], x_dis.shape[0]
    row_c, col_c = circ_edge_index[0], circ_edge_index[1]
    row_d, col_d = dis_edge_index[0], dis_edge_index[1]
    ec, ed = row_c.shape[0], row_d.shape[0]

    ew_c = _sc_edge_gather(circ_adj, row_c, col_c)
    ew_d = _sc_edge_gather(dis_adj, row_d, col_d)

    def _degpad(dp, n):
        z = jnp.zeros((n, 128), jnp.float32)
        return z.at[:, 0].set(dp[:n]), z.at[:, 0].set(dp[n:])

    degp_c = _sc_edge_pass(row_c, col_c, ew_c, None, None, None, 0, nc)
    dc0, dc1 = _degpad(degp_c, nc)
    dinv_c, dsum_c = _tc_deg_fin(dc0, dc1, 1000)
    mean_c = (dsum_c[0, 0] - nc) / ec
    degp_d = _sc_edge_pass(row_d, col_d, ew_d, None, None, None, 0, nd)
    dd0, dd1 = _degpad(degp_d, nd)
    dinv_d, dsum_d = _tc_deg_fin(dd0, dd1, 1024)
    mean_d = (dsum_d[0, 0] - nd) / ed

    x_cir_f1 = _layer(x_cir, row_c, col_c, ew_c, dinv_c, mean_c,
                      params["gcn_cir1"], params["gat_cir1"], 2, 1000)
    x_cir_f2 = _layer(x_cir_f1, row_c, col_c, ew_c, dinv_c, mean_c,
                      params["gcn_cir2"], params["gat_cir2"], 1, 1000)

    x_dis_f1 = _layer(x_dis, row_d, col_d, ew_d, dinv_d, mean_d,
                      params["gcn_dis1"], params["gat_dis1"], 4, 1024)
    x_dis_f2 = _layer(x_dis_f1, row_d, col_d, ew_d, dinv_d, mean_d,
                      params["gcn_dis2"], params["gat_dis1"], 4, 1024)

    dis_fea = _fea_matmul(x_dis_f1, x_dis_f2, params["cnn_dis"]["W"],
                          params["cnn_dis"]["b"], bm=1024)
    cir_fea = _fea_matmul(x_cir_f1, x_cir_f2, params["cnn_cir"]["W"],
                          params["cnn_cir"]["b"], bm=1000)
    scores = _score_matmul(cir_fea, dis_fea, bm=1000, bn=1024)
    return (scores, cir_fea, dis_fea)
